# Initial kernel scaffold; baseline (speedup 1.0000x reference)
import jax
import jax.numpy as jnp
from jax.experimental import pallas as pl


def kernel(density, species, W1, b1, W2, b2):
    raise NotImplementedError("write your pallas kernel here")


# speedup vs baseline: 1.2975x; 1.2975x over previous
import functools

import jax
import jax.numpy as jnp
from jax import lax
from jax.experimental import pallas as pl
from jax.experimental.pallas import tpu as pltpu
from jax.experimental.pallas import tpu_sc as plsc

_SC_CORES = 2
_SC_SUBCORES = 16
_NW = _SC_CORES * _SC_SUBCORES


def _routing(species, N, E, BM, G):
    perm = jnp.argsort(species).astype(jnp.int32)
    counts = jnp.bincount(species, length=E).astype(jnp.int32)
    nblk = (counts + BM - 1) // BM
    blk_end = jnp.cumsum(nblk)
    blk_start = blk_end - nblk
    tok_start = jnp.cumsum(counts) - counts
    g = jnp.arange(G, dtype=jnp.int32)
    e_of_g = jnp.searchsorted(blk_end, g, side="right").astype(jnp.int32)
    e_of_g = jnp.minimum(e_of_g, E - 1)
    i = jnp.arange(BM, dtype=jnp.int32)
    r = (g[:, None] - blk_start[e_of_g][:, None]) * BM + i[None, :]
    tok = tok_start[e_of_g][:, None] + r
    valid = r < counts[e_of_g][:, None]
    gathered_tok = perm[jnp.clip(tok, 0, N - 1)]
    src = jnp.where(valid, gathered_tok, 0)
    slot = g[:, None] * BM + i[None, :]
    dst = jnp.where(valid, gathered_tok, N + slot)
    return e_of_g, src.reshape(-1), dst.reshape(-1)


def _chunk_rows(b_per_w, row_bytes, budget=256 * 1024):
    cmax = max(8, min(b_per_w, budget // row_bytes))
    for c in range(cmax - cmax % 8, 0, -8):
        if b_per_w % c == 0:
            return c
    return 8


def _sc_gather(table, idx, R):
    _, D = table.shape
    b_per_w = R // _NW
    C = _chunk_rows(b_per_w, D * 4)
    mesh = plsc.VectorSubcoreMesh(core_axis_name="c", subcore_axis_name="s")

    @functools.partial(
        pl.kernel,
        mesh=mesh,
        out_type=jax.ShapeDtypeStruct((R, D), table.dtype),
        scratch_types=[
            pltpu.VMEM((C,), jnp.int32),
            pltpu.VMEM((C, D), table.dtype),
            pltpu.SemaphoreType.DMA,
        ],
    )
    def gather_k(table_hbm, idx_hbm, out_hbm, idx_v, rows_v, sem):
        wid = lax.axis_index("s") * _SC_CORES + lax.axis_index("c")
        base = wid * b_per_w

        @pl.loop(0, b_per_w, step=C)
        def _(j):
            pltpu.sync_copy(idx_hbm.at[pl.ds(base + j, C)], idx_v)
            pltpu.async_copy(table_hbm.at[idx_v], rows_v, sem).wait()
            pltpu.sync_copy(rows_v, out_hbm.at[pl.ds(base + j, C)])

    return gather_k(table, idx)


def _sc_scatter(rows, idx, M):
    R, D = rows.shape
    b_per_w = R // _NW
    C = _chunk_rows(b_per_w, D * 4)
    mesh = plsc.VectorSubcoreMesh(core_axis_name="c", subcore_axis_name="s")

    @functools.partial(
        pl.kernel,
        mesh=mesh,
        out_type=jax.ShapeDtypeStruct((M, D), rows.dtype),
        scratch_types=[
            pltpu.VMEM((C,), jnp.int32),
            pltpu.VMEM((C, D), rows.dtype),
            pltpu.SemaphoreType.DMA,
        ],
    )
    def scatter_k(rows_hbm, idx_hbm, out_hbm, idx_v, rows_v, sem):
        wid = lax.axis_index("s") * _SC_CORES + lax.axis_index("c")
        base = wid * b_per_w

        @pl.loop(0, b_per_w, step=C)
        def _(j):
            pltpu.sync_copy(idx_hbm.at[pl.ds(base + j, C)], idx_v)
            pltpu.sync_copy(rows_hbm.at[pl.ds(base + j, C)], rows_v)
            pltpu.async_copy(rows_v, out_hbm.at[idx_v], sem).wait()

    return scatter_k(rows, idx)


def _mlp_body(e_ref, x_ref, w1_ref, b1_ref, w2_ref, b2_ref, y_ref):
    h = jnp.dot(x_ref[...], w1_ref[0], preferred_element_type=jnp.float32)
    h = jax.nn.silu(h + b1_ref[0, 0])
    y = jnp.dot(h, w2_ref[0], preferred_element_type=jnp.float32)
    y_ref[...] = y + b2_ref[0, 0]


def _grouped_mlp(e_of_g, x, W1, b1, W2, b2, BM, G):
    R = x.shape[0]
    D_IN, D_H = W1.shape[1], W1.shape[2]
    D_OUT = W2.shape[2]
    grid_spec = pltpu.PrefetchScalarGridSpec(
        num_scalar_prefetch=1,
        grid=(G,),
        in_specs=[
            pl.BlockSpec((BM, D_IN), lambda g, e: (g, 0)),
            pl.BlockSpec((1, D_IN, D_H), lambda g, e: (e[g], 0, 0)),
            pl.BlockSpec((1, 1, D_H), lambda g, e: (e[g], 0, 0)),
            pl.BlockSpec((1, D_H, D_OUT), lambda g, e: (e[g], 0, 0)),
            pl.BlockSpec((1, 1, D_OUT), lambda g, e: (e[g], 0, 0)),
        ],
        out_specs=pl.BlockSpec((BM, D_OUT), lambda g, e: (g, 0)),
    )
    return pl.pallas_call(
        _mlp_body,
        grid_spec=grid_spec,
        out_shape=jax.ShapeDtypeStruct((R, D_OUT), jnp.float32),
    )(e_of_g, x, W1, b1[:, None, :], W2, b2[:, None, :])


def kernel(density, species, W1, b1, W2, b2):
    N, D_IN = density.shape
    E = W1.shape[0]
    D_OUT = W2.shape[2]
    BM = 256
    G = N // BM + E
    R = G * BM

    e_of_g, src, dst = _routing(species, N, E, BM, G)
    gathered = _sc_gather(density, src, R)
    y = _grouped_mlp(e_of_g, gathered, W1, b1, W2, b2, BM, G)
    out_pad = _sc_scatter(y, dst, N + R)
    return out_pad[:N]
